# baseline (speedup 1.0000x reference)
import functools
import jax
import jax.numpy as jnp
from jax import lax
from jax.experimental import pallas as pl
from jax.experimental.pallas import tpu as pltpu
from jax.experimental.pallas import tpu_sc as plsc

T = 2048
D = 1024
F = 512
E = 64
K = 2
C = 128
EC = E * C

NW = 32
TPW = T // NW
PPW = TPW * K
SPW = EC // NW

NEG = -1e30


def _wid():
    return lax.axis_index("s") * 2 + lax.axis_index("c")


def _mesh():
    return plsc.VectorSubcoreMesh(core_axis_name="c", subcore_axis_name="s")


def _router_body(x_ref, g_ref, out_ref):
    xb = x_ref[...]
    gw = g_ref[...]
    logits = lax.dot_general(gw, xb, (((1,), (1,)), ((), ())),
                             preferred_element_type=jnp.float32)
    out_ref[...] = logits[None]


def _router(x, gate_weight):
    return pl.pallas_call(
        _router_body,
        grid=(NW,),
        in_specs=[
            pl.BlockSpec((TPW, D), lambda i: (i, 0)),
            pl.BlockSpec((E, D), lambda i: (0, 0)),
        ],
        out_specs=pl.BlockSpec((1, E, TPW), lambda i: (i, 0, 0)),
        out_shape=jax.ShapeDtypeStruct((NW, E, TPW), jnp.float32),
    )(x, gate_weight)


def _meta1_body(lg_hbm, pe_hbm, pw_hbm, lp_hbm, hist_hbm,
                lg_v, iv_v, wv_v, hist_v, pe_s, pw_s, lp_s, pec_v):
    w = _wid()
    pltpu.sync_copy(lg_hbm.at[w], lg_v)

    lanes = lax.iota(jnp.int32, 16)
    zero_i = jnp.zeros((16,), jnp.int32)
    zero_f = jnp.zeros((16,), jnp.float32)

    for c in range(TPW // 16):
        def esteep(e, carry):
            m1, i1, m2, i2 = carry
            v = lg_v[e, pl.ds(c * 16, 16)]
            ev = zero_i + e
            gt1 = v > m1
            gt2 = v > m2
            m2n = jnp.where(gt1, m1, jnp.where(gt2, v, m2))
            i2n = jnp.where(gt1, i1, jnp.where(gt2, ev, i2))
            m1n = jnp.where(gt1, v, m1)
            i1n = jnp.where(gt1, ev, i1)
            return m1n, i1n, m2n, i2n
        m1, i1, m2, i2 = lax.fori_loop(
            0, E, esteep,
            (zero_f + NEG, zero_i, zero_f + NEG, zero_i))
        t = jnp.exp(m2 - m1)
        w1 = 1.0 / (1.0 + t)
        w2 = 1.0 - w1
        iv_v[pl.ds(c * 16, 16)] = i1
        iv_v[pl.ds(TPW + c * 16, 16)] = i2
        wv_v[pl.ds(c * 16, 16)] = w1
        wv_v[pl.ds(TPW + c * 16, 16)] = w2

    for q in range(E // 16):
        hist_v[pl.ds(q * 16, 16)] = zero_i

    def pstep(c2, _):
        p_loc = c2 * 16 + lanes
        tok_l = lax.shift_right_logical(p_loc, 1)
        kk = lax.bitwise_and(p_loc, 1)
        flat = kk * TPW + tok_l
        pe_c = plsc.load_gather(iv_v, [flat])
        pw_c = plsc.load_gather(wv_v, [flat])
        base = plsc.load_gather(hist_v, [pe_c])
        pec_v[pl.ds(16, 16)] = pe_c
        dup = zero_i
        tot = zero_i
        for j in range(16):
            b = plsc.load_gather(pec_v, [zero_i + (16 + j)])
            eq = pe_c == b
            dup = dup + jnp.where(eq & (lanes > j), 1, 0)
            tot = tot + jnp.where(eq, 1, 0)
        plsc.store_scatter(hist_v, [pe_c], base + tot)
        pe_s[pl.ds(c2 * 16, 16)] = pe_c
        pw_s[pl.ds(c2 * 16, 16)] = pw_c
        lp_s[pl.ds(c2 * 16, 16)] = base + dup
        return 0

    lax.fori_loop(0, PPW // 16, pstep, 0)

    pltpu.sync_copy(pe_s, pe_hbm.at[pl.ds(w * PPW, PPW)])
    pltpu.sync_copy(pw_s, pw_hbm.at[pl.ds(w * PPW, PPW)])
    pltpu.sync_copy(lp_s, lp_hbm.at[pl.ds(w * PPW, PPW)])
    pltpu.sync_copy(hist_v, hist_hbm.at[w])


def _meta1(logits3d):
    return pl.kernel(
        _meta1_body,
        out_type=[
            jax.ShapeDtypeStruct((T * K,), jnp.int32),
            jax.ShapeDtypeStruct((T * K,), jnp.float32),
            jax.ShapeDtypeStruct((T * K,), jnp.int32),
            jax.ShapeDtypeStruct((NW, E), jnp.int32),
        ],
        mesh=_mesh(),
        compiler_params=pltpu.CompilerParams(needs_layout_passes=False),
        scratch_types=[
            pltpu.VMEM((E, TPW), jnp.float32),
            pltpu.VMEM((K * TPW,), jnp.int32),
            pltpu.VMEM((K * TPW,), jnp.float32),
            pltpu.VMEM((E,), jnp.int32),
            pltpu.VMEM((PPW,), jnp.int32),
            pltpu.VMEM((PPW,), jnp.float32),
            pltpu.VMEM((PPW,), jnp.int32),
            pltpu.VMEM((32,), jnp.int32),
        ],
    )(logits3d)


def _meta2_body(pe_hbm, pw_hbm, lp_hbm, hist_hbm,
                slot_hbm, wv_hbm, tok_hbm,
                hall_v, offs_v, pe_v, pw_v, lp_v,
                slot_s, wv_s, sidx_v, ptv_v, sem):
    w = _wid()
    pltpu.sync_copy(hist_hbm, hall_v)
    pltpu.sync_copy(pe_hbm.at[pl.ds(w * PPW, PPW)], pe_v)
    pltpu.sync_copy(pw_hbm.at[pl.ds(w * PPW, PPW)], pw_v)
    pltpu.sync_copy(lp_hbm.at[pl.ds(w * PPW, PPW)], lp_v)

    lanes = lax.iota(jnp.int32, 16)
    zero_i = jnp.zeros((16,), jnp.int32)

    for q in range(E // 16):
        def hsum(v, acc):
            return acc + hall_v[v, pl.ds(q * 16, 16)]
        offs_v[pl.ds(q * 16, 16)] = lax.fori_loop(0, w, hsum, zero_i)

    def cstep(c, _):
        pe_c = pe_v[pl.ds(c * 16, 16)]
        pw_c = pw_v[pl.ds(c * 16, 16)]
        pos = plsc.load_gather(offs_v, [pe_c]) + lp_v[pl.ds(c * 16, 16)]
        valid = pos < C
        slot = pe_c * C + pos
        slot_s[pl.ds(c * 16, 16)] = jnp.where(valid, slot, EC - 1)
        wv_s[pl.ds(c * 16, 16)] = pw_c * valid.astype(jnp.float32)
        sidx_v[pl.ds(c * 16, 16)] = jnp.where(valid, slot, EC)
        p_glob = w * PPW + c * 16 + lanes
        ptv_v[pl.ds(c * 16, 16)] = lax.shift_right_logical(p_glob, 1)
        return 0

    lax.fori_loop(0, PPW // 16, cstep, 0)

    pltpu.async_copy(ptv_v, tok_hbm.at[sidx_v], sem).wait()
    pltpu.sync_copy(slot_s, slot_hbm.at[pl.ds(w * PPW, PPW)])
    pltpu.sync_copy(wv_s, wv_hbm.at[pl.ds(w * PPW, PPW)])


def _meta2(pe, pw, lp, hist):
    return pl.kernel(
        _meta2_body,
        out_type=[
            jax.ShapeDtypeStruct((T * K,), jnp.int32),
            jax.ShapeDtypeStruct((T * K,), jnp.float32),
            jax.ShapeDtypeStruct((EC + 16,), jnp.int32),
        ],
        mesh=_mesh(),
        compiler_params=pltpu.CompilerParams(needs_layout_passes=False),
        scratch_types=[
            pltpu.VMEM((NW, E), jnp.int32),
            pltpu.VMEM((E,), jnp.int32),
            pltpu.VMEM((PPW,), jnp.int32),
            pltpu.VMEM((PPW,), jnp.float32),
            pltpu.VMEM((PPW,), jnp.int32),
            pltpu.VMEM((PPW,), jnp.int32),
            pltpu.VMEM((PPW,), jnp.float32),
            pltpu.VMEM((PPW,), jnp.int32),
            pltpu.VMEM((PPW,), jnp.int32),
            pltpu.SemaphoreType.DMA,
        ],
    )(pe, pw, lp, hist)


_DCH = 64


def _dispatch_body(tok_hbm, x_hbm, buf_hbm, idx_v, cidx_v, rows_v, sem):
    w = _wid()
    for h in range(SPW // _DCH):
        base = w * SPW + h * _DCH
        pltpu.sync_copy(tok_hbm.at[pl.ds(base, _DCH)], idx_v)
        for q in range(_DCH // 16):
            iv = idx_v[pl.ds(q * 16, 16)]
            cidx_v[pl.ds(q * 16, 16)] = jnp.minimum(
                jnp.maximum(iv, 0), T - 1)
        pltpu.async_copy(x_hbm.at[cidx_v], rows_v, sem).wait()
        pltpu.sync_copy(rows_v, buf_hbm.at[pl.ds(base, _DCH)])


def _dispatch(tok_of_slot, x):
    return pl.kernel(
        _dispatch_body,
        out_type=jax.ShapeDtypeStruct((EC, D), jnp.float32),
        mesh=_mesh(),
        compiler_params=pltpu.CompilerParams(needs_layout_passes=False),
        scratch_types=[
            pltpu.VMEM((_DCH,), jnp.int32),
            pltpu.VMEM((_DCH,), jnp.int32),
            pltpu.VMEM((_DCH, D), jnp.float32),
            pltpu.SemaphoreType.DMA,
        ],
    )(tok_of_slot, x)


def _moe_body(buf_ref, w1_ref, w3_ref, w2_ref, out_ref):
    xb = buf_ref[...]
    g = lax.dot_general(xb, w1_ref[0], (((1,), (1,)), ((), ())),
                        preferred_element_type=jnp.float32)
    u = lax.dot_general(xb, w3_ref[0], (((1,), (1,)), ((), ())),
                        preferred_element_type=jnp.float32)
    h = g * jax.nn.sigmoid(g) * u
    out_ref[...] = lax.dot_general(h, w2_ref[0], (((1,), (1,)), ((), ())),
                                   preferred_element_type=jnp.float32)


def _moe_mlp(buf, w1, w3, w2):
    return pl.pallas_call(
        _moe_body,
        grid=(E,),
        in_specs=[
            pl.BlockSpec((C, D), lambda e: (e, 0)),
            pl.BlockSpec((1, F, D), lambda e: (e, 0, 0)),
            pl.BlockSpec((1, F, D), lambda e: (e, 0, 0)),
            pl.BlockSpec((1, D, F), lambda e: (e, 0, 0)),
        ],
        out_specs=pl.BlockSpec((C, D), lambda e: (e, 0)),
        out_shape=jax.ShapeDtypeStruct((EC, D), jnp.float32),
    )(buf, w1, w3, w2)


_SB = 256


def _shared_body(x_ref, w1_ref, w3_ref, w2_ref, out_ref):
    xb = x_ref[...]
    g = lax.dot_general(xb, w1_ref[...], (((1,), (1,)), ((), ())),
                        preferred_element_type=jnp.float32)
    u = lax.dot_general(xb, w3_ref[...], (((1,), (1,)), ((), ())),
                        preferred_element_type=jnp.float32)
    h = g * jax.nn.sigmoid(g) * u
    out_ref[...] = lax.dot_general(h, w2_ref[...], (((1,), (1,)), ((), ())),
                                   preferred_element_type=jnp.float32)


def _shared_mlp(x, sw1, sw3, sw2):
    return pl.pallas_call(
        _shared_body,
        grid=(T // _SB,),
        in_specs=[
            pl.BlockSpec((_SB, D), lambda i: (i, 0)),
            pl.BlockSpec((F, D), lambda i: (0, 0)),
            pl.BlockSpec((F, D), lambda i: (0, 0)),
            pl.BlockSpec((D, F), lambda i: (0, 0)),
        ],
        out_specs=pl.BlockSpec((_SB, D), lambda i: (i, 0)),
        out_shape=jax.ShapeDtypeStruct((T, D), jnp.float32),
    )(x, sw1, sw3, sw2)


_CCH = 32


def _combine_body(slot_hbm, wv_hbm, ebuf_hbm, sh_hbm, out_hbm,
                  slot_v, wvv_v, grows_v, out_v, sem):
    w = _wid()
    pltpu.sync_copy(slot_hbm.at[pl.ds(w * PPW, PPW)], slot_v)
    pltpu.sync_copy(wv_hbm.at[pl.ds(w * PPW, PPW)], wvv_v)
    for h in range(TPW // _CCH):
        pltpu.async_copy(
            ebuf_hbm.at[slot_v.at[pl.ds(h * _CCH * K, _CCH * K)]],
            grows_v, sem).wait()
        tbase = w * TPW + h * _CCH
        pltpu.sync_copy(sh_hbm.at[pl.ds(tbase, _CCH)], out_v)

        def tstep(r, _):
            pbase = h * _CCH * K + 2 * r
            wb1 = plsc.load_gather(wvv_v, [jnp.zeros((16,), jnp.int32) + pbase])
            wb2 = plsc.load_gather(
                wvv_v, [jnp.zeros((16,), jnp.int32) + (pbase + 1)])

            def vstep(v, _):
                g1 = grows_v[2 * r, pl.ds(v * 16, 16)]
                g2 = grows_v[2 * r + 1, pl.ds(v * 16, 16)]
                acc = out_v[r, pl.ds(v * 16, 16)]
                out_v[r, pl.ds(v * 16, 16)] = acc + wb1 * g1 + wb2 * g2
                return 0

            lax.fori_loop(0, D // 16, vstep, 0)
            return 0

        lax.fori_loop(0, _CCH, tstep, 0)
        pltpu.sync_copy(out_v, out_hbm.at[pl.ds(tbase, _CCH)])


def _combine(slot, wv, ebuf, shared):
    return pl.kernel(
        _combine_body,
        out_type=jax.ShapeDtypeStruct((T, D), jnp.float32),
        mesh=_mesh(),
        compiler_params=pltpu.CompilerParams(needs_layout_passes=False),
        scratch_types=[
            pltpu.VMEM((PPW,), jnp.int32),
            pltpu.VMEM((PPW,), jnp.float32),
            pltpu.VMEM((_CCH * K, D), jnp.float32),
            pltpu.VMEM((_CCH, D), jnp.float32),
            pltpu.SemaphoreType.DMA,
        ],
    )(slot, wv, ebuf, shared)


def kernel(hidden_states, gate_weight, w1, w3, w2, shared_w1, shared_w3,
           shared_w2):
    x = hidden_states
    logits3d = _router(x, gate_weight)
    pe, pw, lp, hist = _meta1(logits3d)
    slot, wv, tok_of_slot = _meta2(pe, pw, lp, hist)
    buf = _dispatch(tok_of_slot, x)
    ebuf = _moe_mlp(buf, w1, w3, w2)
    shared = _shared_mlp(x, shared_w1, shared_w3, shared_w2)
    return _combine(slot, wv, ebuf, shared)


# speedup vs baseline: 1.0145x; 1.0145x over previous
import functools
import jax
import jax.numpy as jnp
from jax import lax
from jax.experimental import pallas as pl
from jax.experimental.pallas import tpu as pltpu
from jax.experimental.pallas import tpu_sc as plsc

T = 2048
D = 1024
F = 512
E = 64
K = 2
C = 128
EC = E * C

NW = 32
TPW = T // NW
PPW = TPW * K
SPW = EC // NW

NEG = -1e30


def _wid():
    return lax.axis_index("s") * 2 + lax.axis_index("c")


def _mesh():
    return plsc.VectorSubcoreMesh(core_axis_name="c", subcore_axis_name="s")


def _router_body(x_ref, g_ref, out_ref):
    xb = x_ref[...]
    gw = g_ref[...]
    logits = lax.dot_general(gw, xb, (((1,), (1,)), ((), ())),
                             preferred_element_type=jnp.float32)
    out_ref[...] = logits[None]


def _router(x, gate_weight):
    return pl.pallas_call(
        _router_body,
        grid=(NW,),
        in_specs=[
            pl.BlockSpec((TPW, D), lambda i: (i, 0)),
            pl.BlockSpec((E, D), lambda i: (0, 0)),
        ],
        out_specs=pl.BlockSpec((1, E, TPW), lambda i: (i, 0, 0)),
        out_shape=jax.ShapeDtypeStruct((NW, E, TPW), jnp.float32),
    )(x, gate_weight)


def _meta1_body(lg_hbm, pe_hbm, pw_hbm, lp_hbm, hist_hbm,
                lg_v, iv_v, wv_v, hist_v, pe_s, pw_s, lp_s, pec_v):
    w = _wid()
    pltpu.sync_copy(lg_hbm.at[w], lg_v)

    lanes = lax.iota(jnp.int32, 16)
    zero_i = jnp.zeros((16,), jnp.int32)
    zero_f = jnp.zeros((16,), jnp.float32)

    for c in range(TPW // 16):
        def esteep(e, carry):
            m1, i1, m2, i2 = carry
            v = lg_v[e, pl.ds(c * 16, 16)]
            ev = zero_i + e
            gt1 = v > m1
            gt2 = v > m2
            m2n = jnp.where(gt1, m1, jnp.where(gt2, v, m2))
            i2n = jnp.where(gt1, i1, jnp.where(gt2, ev, i2))
            m1n = jnp.where(gt1, v, m1)
            i1n = jnp.where(gt1, ev, i1)
            return m1n, i1n, m2n, i2n
        m1, i1, m2, i2 = lax.fori_loop(
            0, E, esteep,
            (zero_f + NEG, zero_i, zero_f + NEG, zero_i))
        t = jnp.exp(m2 - m1)
        w1 = 1.0 / (1.0 + t)
        w2 = 1.0 - w1
        iv_v[pl.ds(c * 16, 16)] = i1
        iv_v[pl.ds(TPW + c * 16, 16)] = i2
        wv_v[pl.ds(c * 16, 16)] = w1
        wv_v[pl.ds(TPW + c * 16, 16)] = w2

    for q in range(E // 16):
        hist_v[pl.ds(q * 16, 16)] = zero_i

    def pstep(c2, _):
        p_loc = c2 * 16 + lanes
        tok_l = lax.shift_right_logical(p_loc, 1)
        kk = lax.bitwise_and(p_loc, 1)
        flat = kk * TPW + tok_l
        pe_c = plsc.load_gather(iv_v, [flat])
        pw_c = plsc.load_gather(wv_v, [flat])
        base = plsc.load_gather(hist_v, [pe_c])
        pec_v[pl.ds(16, 16)] = pe_c
        dup = zero_i
        tot = zero_i
        for j in range(16):
            b = plsc.load_gather(pec_v, [zero_i + (16 + j)])
            eq = pe_c == b
            dup = dup + jnp.where(eq & (lanes > j), 1, 0)
            tot = tot + jnp.where(eq, 1, 0)
        plsc.store_scatter(hist_v, [pe_c], base + tot)
        pe_s[pl.ds(c2 * 16, 16)] = pe_c
        pw_s[pl.ds(c2 * 16, 16)] = pw_c
        lp_s[pl.ds(c2 * 16, 16)] = base + dup
        return 0

    lax.fori_loop(0, PPW // 16, pstep, 0)

    pltpu.sync_copy(pe_s, pe_hbm.at[pl.ds(w * PPW, PPW)])
    pltpu.sync_copy(pw_s, pw_hbm.at[pl.ds(w * PPW, PPW)])
    pltpu.sync_copy(lp_s, lp_hbm.at[pl.ds(w * PPW, PPW)])
    pltpu.sync_copy(hist_v, hist_hbm.at[w])


def _meta1(logits3d):
    return pl.kernel(
        _meta1_body,
        out_type=[
            jax.ShapeDtypeStruct((T * K,), jnp.int32),
            jax.ShapeDtypeStruct((T * K,), jnp.float32),
            jax.ShapeDtypeStruct((T * K,), jnp.int32),
            jax.ShapeDtypeStruct((NW, E), jnp.int32),
        ],
        mesh=_mesh(),
        compiler_params=pltpu.CompilerParams(needs_layout_passes=False),
        scratch_types=[
            pltpu.VMEM((E, TPW), jnp.float32),
            pltpu.VMEM((K * TPW,), jnp.int32),
            pltpu.VMEM((K * TPW,), jnp.float32),
            pltpu.VMEM((E,), jnp.int32),
            pltpu.VMEM((PPW,), jnp.int32),
            pltpu.VMEM((PPW,), jnp.float32),
            pltpu.VMEM((PPW,), jnp.int32),
            pltpu.VMEM((32,), jnp.int32),
        ],
    )(logits3d)


def _meta2_body(pe_hbm, pw_hbm, lp_hbm, hist_hbm,
                slot_hbm, wv_hbm, tok_hbm,
                hall_v, offs_v, pe_v, pw_v, lp_v,
                slot_s, wv_s, sidx_v, ptv_v, sem):
    w = _wid()
    pltpu.sync_copy(hist_hbm, hall_v)
    pltpu.sync_copy(pe_hbm.at[pl.ds(w * PPW, PPW)], pe_v)
    pltpu.sync_copy(pw_hbm.at[pl.ds(w * PPW, PPW)], pw_v)
    pltpu.sync_copy(lp_hbm.at[pl.ds(w * PPW, PPW)], lp_v)

    lanes = lax.iota(jnp.int32, 16)
    zero_i = jnp.zeros((16,), jnp.int32)

    for q in range(E // 16):
        def hsum(v, acc):
            return acc + hall_v[v, pl.ds(q * 16, 16)]
        offs_v[pl.ds(q * 16, 16)] = lax.fori_loop(0, w, hsum, zero_i)

    def cstep(c, _):
        pe_c = pe_v[pl.ds(c * 16, 16)]
        pw_c = pw_v[pl.ds(c * 16, 16)]
        pos = plsc.load_gather(offs_v, [pe_c]) + lp_v[pl.ds(c * 16, 16)]
        valid = pos < C
        slot = pe_c * C + pos
        slot_s[pl.ds(c * 16, 16)] = jnp.where(valid, slot, EC - 1)
        wv_s[pl.ds(c * 16, 16)] = pw_c * valid.astype(jnp.float32)
        sidx_v[pl.ds(c * 16, 16)] = jnp.where(valid, slot, EC)
        p_glob = w * PPW + c * 16 + lanes
        ptv_v[pl.ds(c * 16, 16)] = lax.shift_right_logical(p_glob, 1)
        return 0

    lax.fori_loop(0, PPW // 16, cstep, 0)

    pltpu.async_copy(ptv_v, tok_hbm.at[sidx_v], sem).wait()
    pltpu.sync_copy(slot_s, slot_hbm.at[pl.ds(w * PPW, PPW)])
    pltpu.sync_copy(wv_s, wv_hbm.at[pl.ds(w * PPW, PPW)])


def _meta2(pe, pw, lp, hist):
    return pl.kernel(
        _meta2_body,
        out_type=[
            jax.ShapeDtypeStruct((T * K,), jnp.int32),
            jax.ShapeDtypeStruct((T * K,), jnp.float32),
            jax.ShapeDtypeStruct((EC + 16,), jnp.int32),
        ],
        mesh=_mesh(),
        compiler_params=pltpu.CompilerParams(needs_layout_passes=False),
        scratch_types=[
            pltpu.VMEM((NW, E), jnp.int32),
            pltpu.VMEM((E,), jnp.int32),
            pltpu.VMEM((PPW,), jnp.int32),
            pltpu.VMEM((PPW,), jnp.float32),
            pltpu.VMEM((PPW,), jnp.int32),
            pltpu.VMEM((PPW,), jnp.int32),
            pltpu.VMEM((PPW,), jnp.float32),
            pltpu.VMEM((PPW,), jnp.int32),
            pltpu.VMEM((PPW,), jnp.int32),
            pltpu.SemaphoreType.DMA,
        ],
    )(pe, pw, lp, hist)


_DCH = 32
_DNC = SPW // _DCH


def _dispatch_body(tok_hbm, x_hbm, buf_hbm, idx_v, cidx0, cidx1, rows0,
                   rows1, sem0, sem1):
    w = _wid()
    rows = (rows0, rows1)
    cidx = (cidx0, cidx1)
    sems = (sem0, sem1)

    def stage(h, b):
        pltpu.sync_copy(tok_hbm.at[pl.ds(w * SPW + h * _DCH, _DCH)], idx_v)
        for q in range(_DCH // 16):
            iv = idx_v[pl.ds(q * 16, 16)]
            cidx[b][pl.ds(q * 16, 16)] = jnp.minimum(jnp.maximum(iv, 0), T - 1)
        return pltpu.async_copy(x_hbm.at[cidx[b]], rows[b], sems[b])

    g = stage(0, 0)
    for h in range(_DNC):
        b = h % 2
        g.wait()
        if h + 1 < _DNC:
            g = stage(h + 1, 1 - b)
        pltpu.sync_copy(rows[b], buf_hbm.at[pl.ds(w * SPW + h * _DCH, _DCH)])


def _dispatch(tok_of_slot, x):
    return pl.kernel(
        _dispatch_body,
        out_type=jax.ShapeDtypeStruct((EC, D), jnp.float32),
        mesh=_mesh(),
        compiler_params=pltpu.CompilerParams(needs_layout_passes=False),
        scratch_types=[
            pltpu.VMEM((_DCH,), jnp.int32),
            pltpu.VMEM((_DCH,), jnp.int32),
            pltpu.VMEM((_DCH,), jnp.int32),
            pltpu.VMEM((_DCH, D), jnp.float32),
            pltpu.VMEM((_DCH, D), jnp.float32),
            pltpu.SemaphoreType.DMA,
            pltpu.SemaphoreType.DMA,
        ],
    )(tok_of_slot, x)


def _moe_body(buf_ref, w1_ref, w3_ref, w2_ref, out_ref):
    xb = buf_ref[...]
    g = lax.dot_general(xb, w1_ref[0], (((1,), (1,)), ((), ())),
                        preferred_element_type=jnp.float32)
    u = lax.dot_general(xb, w3_ref[0], (((1,), (1,)), ((), ())),
                        preferred_element_type=jnp.float32)
    h = g * jax.nn.sigmoid(g) * u
    out_ref[...] = lax.dot_general(h, w2_ref[0], (((1,), (1,)), ((), ())),
                                   preferred_element_type=jnp.float32)


def _moe_mlp(buf, w1, w3, w2):
    return pl.pallas_call(
        _moe_body,
        grid=(E,),
        in_specs=[
            pl.BlockSpec((C, D), lambda e: (e, 0)),
            pl.BlockSpec((1, F, D), lambda e: (e, 0, 0)),
            pl.BlockSpec((1, F, D), lambda e: (e, 0, 0)),
            pl.BlockSpec((1, D, F), lambda e: (e, 0, 0)),
        ],
        out_specs=pl.BlockSpec((C, D), lambda e: (e, 0)),
        out_shape=jax.ShapeDtypeStruct((EC, D), jnp.float32),
    )(buf, w1, w3, w2)


_SB = 256


def _shared_body(x_ref, w1_ref, w3_ref, w2_ref, out_ref):
    xb = x_ref[...]
    g = lax.dot_general(xb, w1_ref[...], (((1,), (1,)), ((), ())),
                        preferred_element_type=jnp.float32)
    u = lax.dot_general(xb, w3_ref[...], (((1,), (1,)), ((), ())),
                        preferred_element_type=jnp.float32)
    h = g * jax.nn.sigmoid(g) * u
    out_ref[...] = lax.dot_general(h, w2_ref[...], (((1,), (1,)), ((), ())),
                                   preferred_element_type=jnp.float32)


def _shared_mlp(x, sw1, sw3, sw2):
    return pl.pallas_call(
        _shared_body,
        grid=(T // _SB,),
        in_specs=[
            pl.BlockSpec((_SB, D), lambda i: (i, 0)),
            pl.BlockSpec((F, D), lambda i: (0, 0)),
            pl.BlockSpec((F, D), lambda i: (0, 0)),
            pl.BlockSpec((D, F), lambda i: (0, 0)),
        ],
        out_specs=pl.BlockSpec((_SB, D), lambda i: (i, 0)),
        out_shape=jax.ShapeDtypeStruct((T, D), jnp.float32),
    )(x, sw1, sw3, sw2)


_CCH = 16
_CNC = TPW // _CCH


def _combine_body(slot_hbm, wv_hbm, ebuf_hbm, sh_hbm, out_hbm,
                  slot_v, wvv_v, grows0, grows1, out_v, sem0, sem1):
    w = _wid()
    grows = (grows0, grows1)
    sems = (sem0, sem1)
    pltpu.sync_copy(slot_hbm.at[pl.ds(w * PPW, PPW)], slot_v)
    pltpu.sync_copy(wv_hbm.at[pl.ds(w * PPW, PPW)], wvv_v)

    def stage(h, b):
        return pltpu.async_copy(
            ebuf_hbm.at[slot_v.at[pl.ds(h * _CCH * K, _CCH * K)]],
            grows[b], sems[b])

    g = stage(0, 0)
    for h in range(_CNC):
        b = h % 2
        tbase = w * TPW + h * _CCH
        pltpu.sync_copy(sh_hbm.at[pl.ds(tbase, _CCH)], out_v)
        g.wait()
        if h + 1 < _CNC:
            g = stage(h + 1, 1 - b)

        def tstep(r, _):
            pbase = h * _CCH * K + 2 * r
            wb1 = plsc.load_gather(wvv_v, [jnp.zeros((16,), jnp.int32) + pbase])
            wb2 = plsc.load_gather(
                wvv_v, [jnp.zeros((16,), jnp.int32) + (pbase + 1)])

            def vstep(v, _):
                g1 = grows[b][2 * r, pl.ds(v * 16, 16)]
                g2 = grows[b][2 * r + 1, pl.ds(v * 16, 16)]
                acc = out_v[r, pl.ds(v * 16, 16)]
                out_v[r, pl.ds(v * 16, 16)] = acc + wb1 * g1 + wb2 * g2
                return 0

            lax.fori_loop(0, D // 16, vstep, 0)
            return 0

        lax.fori_loop(0, _CCH, tstep, 0)
        pltpu.sync_copy(out_v, out_hbm.at[pl.ds(tbase, _CCH)])


def _combine(slot, wv, ebuf, shared):
    return pl.kernel(
        _combine_body,
        out_type=jax.ShapeDtypeStruct((T, D), jnp.float32),
        mesh=_mesh(),
        compiler_params=pltpu.CompilerParams(needs_layout_passes=False),
        scratch_types=[
            pltpu.VMEM((PPW,), jnp.int32),
            pltpu.VMEM((PPW,), jnp.float32),
            pltpu.VMEM((_CCH * K, D), jnp.float32),
            pltpu.VMEM((_CCH * K, D), jnp.float32),
            pltpu.VMEM((_CCH, D), jnp.float32),
            pltpu.SemaphoreType.DMA,
            pltpu.SemaphoreType.DMA,
        ],
    )(slot, wv, ebuf, shared)


def kernel(hidden_states, gate_weight, w1, w3, w2, shared_w1, shared_w3,
           shared_w2):
    x = hidden_states
    logits3d = _router(x, gate_weight)
    pe, pw, lp, hist = _meta1(logits3d)
    slot, wv, tok_of_slot = _meta2(pe, pw, lp, hist)
    buf = _dispatch(tok_of_slot, x)
    ebuf = _moe_mlp(buf, w1, w3, w2)
    shared = _shared_mlp(x, shared_w1, shared_w3, shared_w2)
    return _combine(slot, wv, ebuf, shared)


# speedup vs baseline: 1.5240x; 1.5022x over previous
import functools
import jax
import jax.numpy as jnp
from jax import lax
from jax.experimental import pallas as pl
from jax.experimental.pallas import tpu as pltpu
from jax.experimental.pallas import tpu_sc as plsc

T = 2048
D = 1024
F = 512
E = 64
K = 2
C = 128
EC = E * C

NW = 32
TPW = T // NW
PPW = TPW * K
SPW = EC // NW

NEG = -1e30


def _wid():
    return lax.axis_index("s") * 2 + lax.axis_index("c")


def _mesh():
    return plsc.VectorSubcoreMesh(core_axis_name="c", subcore_axis_name="s")


def _router_body(x_ref, g_ref, out_ref):
    xb = x_ref[...]
    gw = g_ref[...]
    logits = lax.dot_general(gw, xb, (((1,), (1,)), ((), ())),
                             preferred_element_type=jnp.float32)
    out_ref[...] = logits[None]


def _router(x, gate_weight):
    return pl.pallas_call(
        _router_body,
        grid=(NW,),
        in_specs=[
            pl.BlockSpec((TPW, D), lambda i: (i, 0)),
            pl.BlockSpec((E, D), lambda i: (0, 0)),
        ],
        out_specs=pl.BlockSpec((1, E, TPW), lambda i: (i, 0, 0)),
        out_shape=jax.ShapeDtypeStruct((NW, E, TPW), jnp.float32),
    )(x, gate_weight)


def _meta1_body(lg_hbm, pe_hbm, pw_hbm, lp_hbm, hist_hbm,
                lg_v, iv_v, wv_v, hist_v, pe_s, pw_s, lp_s, pec_v):
    w = _wid()
    pltpu.sync_copy(lg_hbm.at[w], lg_v)

    lanes = lax.iota(jnp.int32, 16)
    zero_i = jnp.zeros((16,), jnp.int32)
    zero_f = jnp.zeros((16,), jnp.float32)

    for c in range(TPW // 16):
        def esteep(e, carry):
            m1, i1, m2, i2 = carry
            v = lg_v[e, pl.ds(c * 16, 16)]
            ev = zero_i + e
            gt1 = v > m1
            gt2 = v > m2
            m2n = jnp.where(gt1, m1, jnp.where(gt2, v, m2))
            i2n = jnp.where(gt1, i1, jnp.where(gt2, ev, i2))
            m1n = jnp.where(gt1, v, m1)
            i1n = jnp.where(gt1, ev, i1)
            return m1n, i1n, m2n, i2n
        m1, i1, m2, i2 = lax.fori_loop(
            0, E, esteep,
            (zero_f + NEG, zero_i, zero_f + NEG, zero_i))
        t = jnp.exp(m2 - m1)
        w1 = 1.0 / (1.0 + t)
        w2 = 1.0 - w1
        iv_v[pl.ds(c * 16, 16)] = i1
        iv_v[pl.ds(TPW + c * 16, 16)] = i2
        wv_v[pl.ds(c * 16, 16)] = w1
        wv_v[pl.ds(TPW + c * 16, 16)] = w2

    for q in range(E // 16):
        hist_v[pl.ds(q * 16, 16)] = zero_i

    def pstep(c2, _):
        p_loc = c2 * 16 + lanes
        tok_l = lax.shift_right_logical(p_loc, 1)
        kk = lax.bitwise_and(p_loc, 1)
        flat = kk * TPW + tok_l
        pe_c = plsc.load_gather(iv_v, [flat])
        pw_c = plsc.load_gather(wv_v, [flat])
        base = plsc.load_gather(hist_v, [pe_c])
        pec_v[pl.ds(16, 16)] = pe_c
        dup = zero_i
        tot = zero_i
        for j in range(16):
            b = plsc.load_gather(pec_v, [zero_i + (16 + j)])
            eq = pe_c == b
            dup = dup + jnp.where(eq & (lanes > j), 1, 0)
            tot = tot + jnp.where(eq, 1, 0)
        plsc.store_scatter(hist_v, [pe_c], base + tot)
        pe_s[pl.ds(c2 * 16, 16)] = pe_c
        pw_s[pl.ds(c2 * 16, 16)] = pw_c
        lp_s[pl.ds(c2 * 16, 16)] = base + dup
        return 0

    lax.fori_loop(0, PPW // 16, pstep, 0)

    pltpu.sync_copy(pe_s, pe_hbm.at[pl.ds(w * PPW, PPW)])
    pltpu.sync_copy(pw_s, pw_hbm.at[pl.ds(w * PPW, PPW)])
    pltpu.sync_copy(lp_s, lp_hbm.at[pl.ds(w * PPW, PPW)])
    pltpu.sync_copy(hist_v, hist_hbm.at[w])


def _meta1(logits3d):
    return pl.kernel(
        _meta1_body,
        out_type=[
            jax.ShapeDtypeStruct((T * K,), jnp.int32),
            jax.ShapeDtypeStruct((T * K,), jnp.float32),
            jax.ShapeDtypeStruct((T * K,), jnp.int32),
            jax.ShapeDtypeStruct((NW, E), jnp.int32),
        ],
        mesh=_mesh(),
        compiler_params=pltpu.CompilerParams(needs_layout_passes=False),
        scratch_types=[
            pltpu.VMEM((E, TPW), jnp.float32),
            pltpu.VMEM((K * TPW,), jnp.int32),
            pltpu.VMEM((K * TPW,), jnp.float32),
            pltpu.VMEM((E,), jnp.int32),
            pltpu.VMEM((PPW,), jnp.int32),
            pltpu.VMEM((PPW,), jnp.float32),
            pltpu.VMEM((PPW,), jnp.int32),
            pltpu.VMEM((32,), jnp.int32),
        ],
    )(logits3d)


def _meta2_body(pe_hbm, pw_hbm, lp_hbm, hist_hbm,
                slot_hbm, wv_hbm, sk0_hbm, sk1_hbm,
                hall_v, offs_v, pe_v, pw_v, lp_v,
                slot_s, wv_s, sidx_v, sk_s):
    w = _wid()
    pltpu.sync_copy(hist_hbm, hall_v)
    pltpu.sync_copy(pe_hbm.at[pl.ds(w * PPW, PPW)], pe_v)
    pltpu.sync_copy(pw_hbm.at[pl.ds(w * PPW, PPW)], pw_v)
    pltpu.sync_copy(lp_hbm.at[pl.ds(w * PPW, PPW)], lp_v)

    lanes = lax.iota(jnp.int32, 16)
    zero_i = jnp.zeros((16,), jnp.int32)

    for q in range(E // 16):
        def hsum(v, acc):
            return acc + hall_v[v, pl.ds(q * 16, 16)]
        offs_v[pl.ds(q * 16, 16)] = lax.fori_loop(0, w, hsum, zero_i)

    def cstep(c, _):
        pe_c = pe_v[pl.ds(c * 16, 16)]
        pw_c = pw_v[pl.ds(c * 16, 16)]
        pos = plsc.load_gather(offs_v, [pe_c]) + lp_v[pl.ds(c * 16, 16)]
        valid = pos < C
        slot = pe_c * C + pos
        slot_s[pl.ds(c * 16, 16)] = jnp.where(valid, slot, pe_c * C)
        wv_s[pl.ds(c * 16, 16)] = pw_c * valid.astype(jnp.float32)
        sidx_v[pl.ds(c * 16, 16)] = jnp.where(valid, slot, EC)
        return 0

    lax.fori_loop(0, PPW // 16, cstep, 0)

    for q in range(TPW // 16):
        ev = (lanes + q * 16) * 2
        sk_s[pl.ds(q * 16, 16)] = plsc.load_gather(sidx_v, [ev])
        sk_s[pl.ds(TPW + q * 16, 16)] = plsc.load_gather(sidx_v, [ev + 1])

    pltpu.sync_copy(slot_s, slot_hbm.at[pl.ds(w * PPW, PPW)])
    pltpu.sync_copy(wv_s, wv_hbm.at[pl.ds(w * PPW, PPW)])
    pltpu.sync_copy(sk_s.at[pl.ds(0, TPW)], sk0_hbm.at[pl.ds(w * TPW, TPW)])
    pltpu.sync_copy(sk_s.at[pl.ds(TPW, TPW)], sk1_hbm.at[pl.ds(w * TPW, TPW)])


def _meta2(pe, pw, lp, hist):
    return pl.kernel(
        _meta2_body,
        out_type=[
            jax.ShapeDtypeStruct((T * K,), jnp.int32),
            jax.ShapeDtypeStruct((T * K,), jnp.float32),
            jax.ShapeDtypeStruct((T,), jnp.int32),
            jax.ShapeDtypeStruct((T,), jnp.int32),
        ],
        mesh=_mesh(),
        compiler_params=pltpu.CompilerParams(needs_layout_passes=False),
        scratch_types=[
            pltpu.VMEM((NW, E), jnp.int32),
            pltpu.VMEM((E,), jnp.int32),
            pltpu.VMEM((PPW,), jnp.int32),
            pltpu.VMEM((PPW,), jnp.float32),
            pltpu.VMEM((PPW,), jnp.int32),
            pltpu.VMEM((PPW,), jnp.int32),
            pltpu.VMEM((PPW,), jnp.float32),
            pltpu.VMEM((PPW,), jnp.int32),
            pltpu.VMEM((PPW,), jnp.int32),
        ],
    )(pe, pw, lp, hist)


def _dispatch_body(sk0_hbm, sk1_hbm, x_hbm, buf_hbm,
                   i0_v, i1_v, rows_v, sem0, sem1):
    w = _wid()
    pltpu.sync_copy(sk0_hbm.at[pl.ds(w * TPW, TPW)], i0_v)
    pltpu.sync_copy(sk1_hbm.at[pl.ds(w * TPW, TPW)], i1_v)
    pltpu.sync_copy(x_hbm.at[pl.ds(w * TPW, TPW)], rows_v)
    d0 = pltpu.async_copy(rows_v, buf_hbm.at[i0_v], sem0)
    d1 = pltpu.async_copy(rows_v, buf_hbm.at[i1_v], sem1)
    d0.wait()
    d1.wait()


def _dispatch(sk0, sk1, x):
    return pl.kernel(
        _dispatch_body,
        out_type=jax.ShapeDtypeStruct((EC + C, D), jnp.float32),
        mesh=_mesh(),
        compiler_params=pltpu.CompilerParams(needs_layout_passes=False),
        scratch_types=[
            pltpu.VMEM((TPW,), jnp.int32),
            pltpu.VMEM((TPW,), jnp.int32),
            pltpu.VMEM((TPW, D), jnp.float32),
            pltpu.SemaphoreType.DMA,
            pltpu.SemaphoreType.DMA,
        ],
    )(sk0, sk1, x)


def _moe_body(buf_ref, w1_ref, w3_ref, w2_ref, out_ref):
    xb = buf_ref[...]
    g = lax.dot_general(xb, w1_ref[0], (((1,), (1,)), ((), ())),
                        preferred_element_type=jnp.float32)
    u = lax.dot_general(xb, w3_ref[0], (((1,), (1,)), ((), ())),
                        preferred_element_type=jnp.float32)
    h = g * jax.nn.sigmoid(g) * u
    out_ref[...] = lax.dot_general(h, w2_ref[0], (((1,), (1,)), ((), ())),
                                   preferred_element_type=jnp.float32)


def _moe_mlp(buf, w1, w3, w2):
    return pl.pallas_call(
        _moe_body,
        grid=(E,),
        in_specs=[
            pl.BlockSpec((C, D), lambda e: (e, 0)),
            pl.BlockSpec((1, F, D), lambda e: (e, 0, 0)),
            pl.BlockSpec((1, F, D), lambda e: (e, 0, 0)),
            pl.BlockSpec((1, D, F), lambda e: (e, 0, 0)),
        ],
        out_specs=pl.BlockSpec((C, D), lambda e: (e, 0)),
        out_shape=jax.ShapeDtypeStruct((EC, D), jnp.float32),
    )(buf, w1, w3, w2)


_SB = 256


def _shared_body(x_ref, w1_ref, w3_ref, w2_ref, out_ref):
    xb = x_ref[...]
    g = lax.dot_general(xb, w1_ref[...], (((1,), (1,)), ((), ())),
                        preferred_element_type=jnp.float32)
    u = lax.dot_general(xb, w3_ref[...], (((1,), (1,)), ((), ())),
                        preferred_element_type=jnp.float32)
    h = g * jax.nn.sigmoid(g) * u
    out_ref[...] = lax.dot_general(h, w2_ref[...], (((1,), (1,)), ((), ())),
                                   preferred_element_type=jnp.float32)


def _shared_mlp(x, sw1, sw3, sw2):
    return pl.pallas_call(
        _shared_body,
        grid=(T // _SB,),
        in_specs=[
            pl.BlockSpec((_SB, D), lambda i: (i, 0)),
            pl.BlockSpec((F, D), lambda i: (0, 0)),
            pl.BlockSpec((F, D), lambda i: (0, 0)),
            pl.BlockSpec((D, F), lambda i: (0, 0)),
        ],
        out_specs=pl.BlockSpec((_SB, D), lambda i: (i, 0)),
        out_shape=jax.ShapeDtypeStruct((T, D), jnp.float32),
    )(x, sw1, sw3, sw2)


_CCH = 16
_CNC = TPW // _CCH


def _combine_body(slot_hbm, wv_hbm, ebuf_hbm, sh_hbm, out_hbm,
                  slot_v, wvv_v, grows0, grows1, out_v, sem0, sem1):
    w = _wid()
    grows = (grows0, grows1)
    sems = (sem0, sem1)
    pltpu.sync_copy(slot_hbm.at[pl.ds(w * PPW, PPW)], slot_v)
    pltpu.sync_copy(wv_hbm.at[pl.ds(w * PPW, PPW)], wvv_v)

    def stage(h, b):
        return pltpu.async_copy(
            ebuf_hbm.at[slot_v.at[pl.ds(h * _CCH * K, _CCH * K)]],
            grows[b], sems[b])

    g = stage(0, 0)
    for h in range(_CNC):
        b = h % 2
        tbase = w * TPW + h * _CCH
        pltpu.sync_copy(sh_hbm.at[pl.ds(tbase, _CCH)], out_v)
        g.wait()
        if h + 1 < _CNC:
            g = stage(h + 1, 1 - b)

        def tstep(r, _):
            pbase = h * _CCH * K + 2 * r
            wb1 = plsc.load_gather(wvv_v, [jnp.zeros((16,), jnp.int32) + pbase])
            wb2 = plsc.load_gather(
                wvv_v, [jnp.zeros((16,), jnp.int32) + (pbase + 1)])

            def vstep(v, _):
                g1 = grows[b][2 * r, pl.ds(v * 16, 16)]
                g2 = grows[b][2 * r + 1, pl.ds(v * 16, 16)]
                acc = out_v[r, pl.ds(v * 16, 16)]
                out_v[r, pl.ds(v * 16, 16)] = acc + wb1 * g1 + wb2 * g2
                return 0

            lax.fori_loop(0, D // 16, vstep, 0)
            return 0

        lax.fori_loop(0, _CCH, tstep, 0)
        pltpu.sync_copy(out_v, out_hbm.at[pl.ds(tbase, _CCH)])


def _combine(slot, wv, ebuf, shared):
    return pl.kernel(
        _combine_body,
        out_type=jax.ShapeDtypeStruct((T, D), jnp.float32),
        mesh=_mesh(),
        compiler_params=pltpu.CompilerParams(needs_layout_passes=False),
        scratch_types=[
            pltpu.VMEM((PPW,), jnp.int32),
            pltpu.VMEM((PPW,), jnp.float32),
            pltpu.VMEM((_CCH * K, D), jnp.float32),
            pltpu.VMEM((_CCH * K, D), jnp.float32),
            pltpu.VMEM((_CCH, D), jnp.float32),
            pltpu.SemaphoreType.DMA,
            pltpu.SemaphoreType.DMA,
        ],
    )(slot, wv, ebuf, shared)


def kernel(hidden_states, gate_weight, w1, w3, w2, shared_w1, shared_w3,
           shared_w2):
    x = hidden_states
    logits3d = _router(x, gate_weight)
    pe, pw, lp, hist = _meta1(logits3d)
    slot, wv, sk0, sk1 = _meta2(pe, pw, lp, hist)
    buf = _dispatch(sk0, sk1, x)
    ebuf = _moe_mlp(buf, w1, w3, w2)
    shared = _shared_mlp(x, shared_w1, shared_w3, shared_w2)
    return _combine(slot, wv, ebuf, shared)
